# Initial kernel scaffold; baseline (speedup 1.0000x reference)
import jax
import jax.numpy as jnp
from jax.experimental import pallas as pl


def kernel(logits, omega):
    raise NotImplementedError("write your pallas kernel here")


# speedup vs baseline: 1.3308x; 1.3308x over previous
import jax
import jax.numpy as jnp
from jax.experimental import pallas as pl

_JITTER_EPS = 0.1
_TOK_BLK = 512


def _mixer_body(lg_ref, om_ref, sample_ref, mult_ref):
    lg = lg_ref[...]
    mx = jnp.max(lg, axis=-1, keepdims=True)
    ids = jax.lax.broadcasted_iota(jnp.int32, lg.shape, 1)
    amax = jnp.min(jnp.where(lg == mx, ids, lg.shape[1]), axis=-1, keepdims=True)
    factor = jnp.maximum(jnp.abs(lg), mx)
    mask = (mx - lg) / factor > 2.0 * _JITTER_EPS
    e = jnp.where(mask, 0.0, jnp.exp(lg - mx))
    m = 1.0 / jnp.sum(e, axis=-1, keepdims=True)
    sample_ref[...] = amax
    mult_ref[...] = m * om_ref[...][None, :]


def kernel(logits, omega):
    n_tok, n_exp = logits.shape
    dim = omega.shape[0]
    grid = (n_tok // _TOK_BLK,)
    sample, multiplier = pl.pallas_call(
        _mixer_body,
        grid=grid,
        in_specs=[
            pl.BlockSpec((_TOK_BLK, n_exp), lambda i: (i, 0)),
            pl.BlockSpec((dim,), lambda i: (0,)),
        ],
        out_specs=[
            pl.BlockSpec((_TOK_BLK, 1), lambda i: (i, 0)),
            pl.BlockSpec((_TOK_BLK, dim), lambda i: (i, 0)),
        ],
        out_shape=[
            jax.ShapeDtypeStruct((n_tok, 1), jnp.int32),
            jax.ShapeDtypeStruct((n_tok, dim), jnp.float32),
        ],
    )(logits, omega)
    return sample, multiplier, jnp.float32(0.0)
